# baseline (speedup 1.0000x reference)
import functools

import jax
import jax.numpy as jnp
from jax.experimental import pallas as pl
from jax.experimental.pallas import tpu as pltpu

_GROUPS = (
    (2, (0, 1, 3, 4)),
    (6, (5, 7, 8)),
    (10, (9, 11, 12)),
    (14, (13, 15, 16)),
)

_N_ROWS = 1048576
_C = 17
_BLOCK_ROWS = 16384
_GRID = _N_ROWS // _BLOCK_ROWS


def _body(x_ref, a_ref, b_ref, out_ref):
    out_ref[...] = (
        jnp.dot(x_ref[...], a_ref[...], preferred_element_type=jnp.float32)
        + b_ref[...]
    )


@jax.jit
def _run(x, a, b):
    return pl.pallas_call(
        _body,
        grid=(_GRID,),
        in_specs=[
            pl.BlockSpec((_BLOCK_ROWS, _C), lambda i: (i, 0)),
            pl.BlockSpec((_C, _C), lambda i: (0, 0)),
            pl.BlockSpec((1, _C), lambda i: (0, 0)),
        ],
        out_specs=pl.BlockSpec((_BLOCK_ROWS, _C), lambda i: (i, 0)),
        out_shape=jax.ShapeDtypeStruct((_N_ROWS, _C), jnp.float32),
    )(x, a, b)


def kernel(x, mu_y, si_y):
    eye = jnp.eye(_C, dtype=jnp.float32)
    a = eye
    b = jnp.zeros((_C,), dtype=jnp.float32)
    for r, members in _GROUPS:
        inv = 1.0 / si_y[r]
        col = jnp.zeros((_C,), dtype=jnp.float32)
        musum = mu_y[r]
        for c in members:
            col = col.at[c].set(-si_y[c] * inv)
            musum = musum + mu_y[c]
        a = a.at[:, r].set(col)
        b = b.at[r].set(-musum * inv)
    return _run(x, a, b.reshape(1, _C))
